# Initial kernel scaffold; baseline (speedup 1.0000x reference)
import functools

import jax
import jax.numpy as jnp
from jax import lax
from jax.experimental import pallas as pl
from jax.experimental.pallas import tpu as pltpu
from jax.experimental.pallas import tpu_sc as plsc

BATCH = 16384
NT = 50
DIM = 32
TOTAL = BATCH * NT
NUM_CORES = 2
NUM_SUBCORES = 16
NW = NUM_CORES * NUM_SUBCORES
PER_W = TOTAL // NW
CHUNK = 1280
NCHUNK = PER_W // CHUNK

_mesh = plsc.VectorSubcoreMesh(
    core_axis_name="c", subcore_axis_name="s",
    num_cores=NUM_CORES, num_subcores=NUM_SUBCORES)


@functools.partial(
    pl.kernel,
    out_type=jax.ShapeDtypeStruct((TOTAL, DIM), jnp.float32),
    mesh=_mesh,
    scratch_types=[
        pltpu.VMEM((CHUNK,), jnp.int32),
        pltpu.VMEM((CHUNK, DIM), jnp.float32),
        pltpu.SemaphoreType.DMA,
    ],
)
def _gather_kernel(idx_hbm, table_hbm, out_hbm, idx_v, rows_v, sem):
    wid = lax.axis_index("s") * NUM_CORES + lax.axis_index("c")
    base = wid * PER_W

    @pl.loop(0, NCHUNK)
    def _step(i):
        off = base + i * CHUNK
        pltpu.sync_copy(idx_hbm.at[pl.ds(off, CHUNK)], idx_v)
        pltpu.async_copy(table_hbm.at[idx_v], rows_v, sem).wait()
        pltpu.sync_copy(rows_v, out_hbm.at[pl.ds(off, CHUNK)])


def kernel(text, table):
    idx = text.reshape(TOTAL)
    out = _gather_kernel(idx, table)
    return out.reshape(BATCH, NT, DIM)


# speedup vs baseline: 1.0988x; 1.0988x over previous
import functools

import jax
import jax.numpy as jnp
from jax import lax
from jax.experimental import pallas as pl
from jax.experimental.pallas import tpu as pltpu
from jax.experimental.pallas import tpu_sc as plsc

BATCH = 16384
NT = 50
DIM = 32
TOTAL = BATCH * NT
NUM_CORES = 2
NUM_SUBCORES = 16
NW = NUM_CORES * NUM_SUBCORES
PER_W = TOTAL // NW
CHUNK = 1280
NCHUNK = PER_W // CHUNK

_mesh = plsc.VectorSubcoreMesh(
    core_axis_name="c", subcore_axis_name="s",
    num_cores=NUM_CORES, num_subcores=NUM_SUBCORES)


@functools.partial(
    pl.kernel,
    out_type=jax.ShapeDtypeStruct((TOTAL, DIM), jnp.float32),
    mesh=_mesh,
    compiler_params=pltpu.CompilerParams(use_tc_tiling_on_sc=False),
    scratch_types=[
        pltpu.VMEM((CHUNK,), jnp.int32),
        pltpu.VMEM((CHUNK, DIM), jnp.float32),
        pltpu.SemaphoreType.DMA,
    ],
)
def _gather_kernel(idx_hbm, table_hbm, out_hbm, idx_v, rows_v, sem):
    wid = lax.axis_index("s") * NUM_CORES + lax.axis_index("c")
    base = wid * PER_W

    @pl.loop(0, NCHUNK)
    def _step(i):
        off = base + i * CHUNK
        pltpu.sync_copy(idx_hbm.at[pl.ds(off, CHUNK)], idx_v)
        pltpu.async_copy(table_hbm.at[idx_v], rows_v, sem).wait()
        pltpu.sync_copy(rows_v, out_hbm.at[pl.ds(off, CHUNK)])


def kernel(text, table):
    idx = text.reshape(TOTAL)
    out = _gather_kernel(idx, table)
    return out.reshape(BATCH, NT, DIM)


# speedup vs baseline: 1.2810x; 1.1658x over previous
import functools

import jax
import jax.numpy as jnp
from jax import lax
from jax.experimental import pallas as pl
from jax.experimental.pallas import tpu as pltpu
from jax.experimental.pallas import tpu_sc as plsc

BATCH = 16384
NT = 50
DIM = 32
NUM_CORES = 2
NUM_SUBCORES = 16
NW = NUM_CORES * NUM_SUBCORES
LANES = 128
BT_TILES = BATCH // LANES
BT_PER_W = BT_TILES // NW
UNITS = BT_PER_W * NT

_mesh = plsc.VectorSubcoreMesh(
    core_axis_name="c", subcore_axis_name="s",
    num_cores=NUM_CORES, num_subcores=NUM_SUBCORES)


@functools.partial(
    pl.kernel,
    out_type=jax.ShapeDtypeStruct((NT, DIM // 8, BT_TILES, 8, LANES),
                                  jnp.float32),
    mesh=_mesh,
    compiler_params=pltpu.CompilerParams(use_tc_tiling_on_sc=False,
                                         needs_layout_passes=False),
    scratch_types=[
        pltpu.VMEM((LANES,), jnp.int32),
        pltpu.VMEM((LANES, DIM), jnp.float32),
        pltpu.VMEM((DIM, LANES), jnp.float32),
        pltpu.SemaphoreType.DMA,
    ],
)
def _gather_kernel(textT_hbm, table_hbm, out_hbm, idx_v, rows_v, tbuf_v, sem):
    wid = lax.axis_index("s") * NUM_CORES + lax.axis_index("c")
    iota = lax.iota(jnp.int32, 16)
    lane_ids = [iota + 16 * l0 for l0 in range(8)]

    @pl.loop(0, UNITS)
    def _unit(u):
        t = u % NT
        bt = wid * BT_PER_W + u // NT
        pltpu.sync_copy(textT_hbm.at[t, pl.ds(bt * LANES, LANES)], idx_v)
        pltpu.async_copy(table_hbm.at[idx_v], rows_v, sem).wait()

        @pl.loop(0, DIM)
        def _transpose(r):
            col = jnp.full((16,), r, jnp.int32)
            for l0 in range(8):
                v = plsc.load_gather(rows_v, [lane_ids[l0], col])
                tbuf_v[r, pl.ds(16 * l0, 16)] = v

        for rt in range(DIM // 8):
            pltpu.sync_copy(tbuf_v.at[pl.ds(8 * rt, 8), :],
                            out_hbm.at[t, rt, bt])


def kernel(text, table):
    textT = text.T
    out5 = _gather_kernel(textT, table)
    return out5.transpose(2, 4, 0, 1, 3).reshape(BATCH, NT, DIM)


# speedup vs baseline: 1.6427x; 1.2824x over previous
import functools

import jax
import jax.numpy as jnp
from jax import lax
from jax.experimental import pallas as pl
from jax.experimental.pallas import tpu as pltpu
from jax.experimental.pallas import tpu_sc as plsc

BATCH = 16384
NT = 50
DIM = 32
NUM_CORES = 2
NUM_SUBCORES = 16
NW = NUM_CORES * NUM_SUBCORES
LANES = 128
BT_TILES = BATCH // LANES
BT_PER_W = BT_TILES // NW
UNITS = BT_PER_W * NT
NBUF = 4

_mesh = plsc.VectorSubcoreMesh(
    core_axis_name="c", subcore_axis_name="s",
    num_cores=NUM_CORES, num_subcores=NUM_SUBCORES)


@functools.partial(
    pl.kernel,
    out_type=jax.ShapeDtypeStruct((NT, DIM // 8, BT_TILES, 8, LANES),
                                  jnp.float32),
    mesh=_mesh,
    compiler_params=pltpu.CompilerParams(use_tc_tiling_on_sc=False,
                                         needs_layout_passes=False),
    scratch_types=[
        pltpu.VMEM((BT_PER_W, NT, LANES), jnp.int32),
        pltpu.VMEM((NBUF, LANES, DIM), jnp.float32),
        pltpu.VMEM((2, DIM // 8, 8, LANES), jnp.float32),
        pltpu.SemaphoreType.DMA,
        pltpu.SemaphoreType.DMA,
        pltpu.SemaphoreType.DMA,
    ],
)
def _gather_kernel(textT_hbm, table_hbm, out_hbm, idx_v, rows_v, tbuf_v,
                   gsem, wsem0, wsem1):
    wid = lax.axis_index("s") * NUM_CORES + lax.axis_index("c")
    iota = lax.iota(jnp.int32, 16)
    lane_ids = [iota + 16 * l0 for l0 in range(8)]

    for bti in range(BT_PER_W):
        bt = wid * BT_PER_W + bti
        pltpu.sync_copy(textT_hbm.at[:, pl.ds(bt * LANES, LANES)],
                        idx_v.at[bti])

    def fire_gather(u):
        t = u % NT
        bti = u // NT
        pltpu.async_copy(table_hbm.at[idx_v.at[bti, t]],
                         rows_v.at[u % NBUF], gsem)

    def wait_gather(u):
        t = u % NT
        bti = u // NT
        pltpu.make_async_copy(table_hbm.at[idx_v.at[bti, t]],
                              rows_v.at[u % NBUF], gsem).wait()

    def out_slot(u):
        t = u % NT
        bt = wid * BT_PER_W + u // NT
        return out_hbm.at[t, :, bt]

    for u in range(NBUF - 1):
        fire_gather(u)

    @pl.loop(0, UNITS)
    def _unit(u):
        @pl.when(u + NBUF - 1 < UNITS)
        def _():
            fire_gather(u + NBUF - 1)
        wait_gather(u)
        rows = rows_v.at[u % NBUF]
        tbuf = tbuf_v.at[u % 2]

        @pl.when((u >= 2) & (u % 2 == 0))
        def _():
            pltpu.make_async_copy(tbuf_v.at[0], out_slot(u), wsem0).wait()

        @pl.when((u >= 2) & (u % 2 == 1))
        def _():
            pltpu.make_async_copy(tbuf_v.at[1], out_slot(u), wsem1).wait()

        @pl.loop(0, 8)
        def _transpose(rg):
            for dr in range(4):
                r = rg * 4 + dr
                col = jnp.full((16,), r, jnp.int32)
                rt = r // 8
                s = r % 8
                for l0 in range(8):
                    v = plsc.load_gather(rows, [lane_ids[l0], col])
                    tbuf[rt, s, pl.ds(16 * l0, 16)] = v

        @pl.when(u % 2 == 0)
        def _():
            pltpu.async_copy(tbuf_v.at[0], out_slot(u), wsem0)

        @pl.when(u % 2 == 1)
        def _():
            pltpu.async_copy(tbuf_v.at[1], out_slot(u), wsem1)

    pltpu.make_async_copy(tbuf_v.at[0], out_hbm.at[0, :, 0], wsem0).wait()
    pltpu.make_async_copy(tbuf_v.at[1], out_hbm.at[0, :, 0], wsem1).wait()


def kernel(text, table):
    textT = text.T
    out5 = _gather_kernel(textT, table)
    return out5.transpose(2, 4, 0, 1, 3).reshape(BATCH, NT, DIM)


# speedup vs baseline: 1.6428x; 1.0000x over previous
import functools

import jax
import jax.numpy as jnp
from jax import lax
from jax.experimental import pallas as pl
from jax.experimental.pallas import tpu as pltpu
from jax.experimental.pallas import tpu_sc as plsc

BATCH = 16384
NT = 50
DIM = 32
NUM_CORES = 2
NUM_SUBCORES = 16
NW = NUM_CORES * NUM_SUBCORES
LANES = 128
BT_TILES = BATCH // LANES
BT_PER_W = BT_TILES // NW
UNITS = BT_PER_W * NT
NBUF = 8

_mesh = plsc.VectorSubcoreMesh(
    core_axis_name="c", subcore_axis_name="s",
    num_cores=NUM_CORES, num_subcores=NUM_SUBCORES)


@functools.partial(
    pl.kernel,
    out_type=jax.ShapeDtypeStruct((NT, DIM // 8, BT_TILES, 8, LANES),
                                  jnp.float32),
    mesh=_mesh,
    compiler_params=pltpu.CompilerParams(use_tc_tiling_on_sc=False,
                                         needs_layout_passes=False),
    scratch_types=[
        pltpu.VMEM((BT_PER_W, NT, LANES), jnp.int32),
        pltpu.VMEM((NBUF, LANES, DIM), jnp.float32),
        pltpu.VMEM((2, DIM // 8, 8, LANES), jnp.float32),
        pltpu.SemaphoreType.DMA,
        pltpu.SemaphoreType.DMA,
        pltpu.SemaphoreType.DMA,
    ],
)
def _gather_kernel(textT_hbm, table_hbm, out_hbm, idx_v, rows_v, tbuf_v,
                   gsem, wsem0, wsem1):
    wid = lax.axis_index("s") * NUM_CORES + lax.axis_index("c")
    iota = lax.iota(jnp.int32, 16)
    lane_ids = [iota + 16 * l0 for l0 in range(8)]

    for bti in range(BT_PER_W):
        bt = wid * BT_PER_W + bti
        pltpu.sync_copy(textT_hbm.at[:, pl.ds(bt * LANES, LANES)],
                        idx_v.at[bti])

    def fire_gather(u):
        t = u % NT
        bti = u // NT
        pltpu.async_copy(table_hbm.at[idx_v.at[bti, t]],
                         rows_v.at[u % NBUF], gsem)

    def wait_gather(u):
        t = u % NT
        bti = u // NT
        pltpu.make_async_copy(table_hbm.at[idx_v.at[bti, t]],
                              rows_v.at[u % NBUF], gsem).wait()

    def out_slot(u):
        t = u % NT
        bt = wid * BT_PER_W + u // NT
        return out_hbm.at[t, :, bt]

    for u in range(NBUF - 1):
        fire_gather(u)

    @pl.loop(0, UNITS)
    def _unit(u):
        @pl.when(u + NBUF - 1 < UNITS)
        def _():
            fire_gather(u + NBUF - 1)
        wait_gather(u)
        rows = rows_v.at[u % NBUF]
        tbuf = tbuf_v.at[u % 2]

        @pl.when((u >= 2) & (u % 2 == 0))
        def _():
            pltpu.make_async_copy(tbuf_v.at[0], out_slot(u), wsem0).wait()

        @pl.when((u >= 2) & (u % 2 == 1))
        def _():
            pltpu.make_async_copy(tbuf_v.at[1], out_slot(u), wsem1).wait()

        @pl.loop(0, 8)
        def _transpose(rg):
            for dr in range(4):
                r = rg * 4 + dr
                col = jnp.full((16,), r, jnp.int32)
                rt = r // 8
                s = r % 8
                for l0 in range(8):
                    v = plsc.load_gather(rows, [lane_ids[l0], col])
                    tbuf[rt, s, pl.ds(16 * l0, 16)] = v

        @pl.when(u % 2 == 0)
        def _():
            pltpu.async_copy(tbuf_v.at[0], out_slot(u), wsem0)

        @pl.when(u % 2 == 1)
        def _():
            pltpu.async_copy(tbuf_v.at[1], out_slot(u), wsem1)

    pltpu.make_async_copy(tbuf_v.at[0], out_hbm.at[0, :, 0], wsem0).wait()
    pltpu.make_async_copy(tbuf_v.at[1], out_hbm.at[0, :, 0], wsem1).wait()


def kernel(text, table):
    textT = text.T
    out5 = _gather_kernel(textT, table)
    return out5.transpose(2, 4, 0, 1, 3).reshape(BATCH, NT, DIM)


# speedup vs baseline: 2.2053x; 1.3424x over previous
import functools

import jax
import jax.numpy as jnp
from jax import lax
from jax.experimental import pallas as pl
from jax.experimental.pallas import tpu as pltpu
from jax.experimental.pallas import tpu_sc as plsc

BATCH = 16384
NT = 50
DIM = 32
NUM_CORES = 2
NUM_SUBCORES = 16
NW = NUM_CORES * NUM_SUBCORES
LANES = 128
BT_TILES = BATCH // LANES
BT_PER_W = BT_TILES // NW
UNITS = BT_PER_W * NT
NBUF = 8

_mesh = plsc.VectorSubcoreMesh(
    core_axis_name="c", subcore_axis_name="s",
    num_cores=NUM_CORES, num_subcores=NUM_SUBCORES)


@functools.partial(
    pl.kernel,
    out_type=jax.ShapeDtypeStruct((NT, DIM // 8, BT_TILES, 8, LANES),
                                  jnp.float32),
    mesh=_mesh,
    compiler_params=pltpu.CompilerParams(use_tc_tiling_on_sc=False,
                                         needs_layout_passes=False),
    scratch_types=[
        pltpu.VMEM((BT_PER_W, NT, LANES), jnp.int32),
        pltpu.VMEM((NBUF, LANES, DIM), jnp.float32),
        pltpu.VMEM((2, DIM // 8, 8, LANES), jnp.float32),
        pltpu.SemaphoreType.DMA,
        pltpu.SemaphoreType.DMA,
        pltpu.SemaphoreType.DMA,
    ],
)
def _gather_kernel(textT_hbm, table_hbm, out_hbm, idx_v, rows_v, tbuf_v,
                   gsem, wsem0, wsem1):
    wid = lax.axis_index("s") * NUM_CORES + lax.axis_index("c")
    iota = lax.iota(jnp.int32, 16)
    lane_ids = [iota + 16 * l0 for l0 in range(8)]

    for bti in range(BT_PER_W):
        bt = wid * BT_PER_W + bti
        pltpu.sync_copy(textT_hbm.at[:, pl.ds(bt * LANES, LANES)],
                        idx_v.at[bti])

    def fire_gather(u):
        t = u % NT
        bti = u // NT
        pltpu.async_copy(table_hbm.at[idx_v.at[bti, t]],
                         rows_v.at[u % NBUF], gsem)

    def wait_gather(u):
        t = u % NT
        bti = u // NT
        pltpu.make_async_copy(table_hbm.at[idx_v.at[bti, t]],
                              rows_v.at[u % NBUF], gsem).wait()

    def out_slot(u):
        t = u % NT
        bt = wid * BT_PER_W + u // NT
        return out_hbm.at[t, :, bt]

    for u in range(NBUF - 1):
        fire_gather(u)

    @pl.loop(0, UNITS)
    def _unit(u):
        @pl.when(u + NBUF - 1 < UNITS)
        def _():
            fire_gather(u + NBUF - 1)
        wait_gather(u)
        rows = rows_v.at[u % NBUF]
        tbuf = tbuf_v.at[u % 2]

        @pl.when((u >= 2) & (u % 2 == 0))
        def _():
            pltpu.make_async_copy(tbuf_v.at[0], out_slot(u), wsem0).wait()

        @pl.when((u >= 2) & (u % 2 == 1))
        def _():
            pltpu.make_async_copy(tbuf_v.at[1], out_slot(u), wsem1).wait()

        @plsc.parallel_loop(0, DIM, unroll=4)
        def _transpose(r):
            col = jnp.full((16,), r, jnp.int32)
            rt = r // 8
            s = r % 8
            for l0 in range(8):
                v = plsc.load_gather(rows, [lane_ids[l0], col])
                tbuf[rt, s, pl.ds(16 * l0, 16)] = v

        @pl.when(u % 2 == 0)
        def _():
            pltpu.async_copy(tbuf_v.at[0], out_slot(u), wsem0)

        @pl.when(u % 2 == 1)
        def _():
            pltpu.async_copy(tbuf_v.at[1], out_slot(u), wsem1)

    pltpu.make_async_copy(tbuf_v.at[0], out_hbm.at[0, :, 0], wsem0).wait()
    pltpu.make_async_copy(tbuf_v.at[1], out_hbm.at[0, :, 0], wsem1).wait()


def kernel(text, table):
    textT = text.T
    out5 = _gather_kernel(textT, table)
    return out5.transpose(2, 4, 0, 1, 3).reshape(BATCH, NT, DIM)
